# Initial kernel scaffold; baseline (speedup 1.0000x reference)
import jax
import jax.numpy as jnp
from jax.experimental import pallas as pl


def kernel(x, I_i):
    raise NotImplementedError("write your pallas kernel here")


# speedup vs baseline: 19.4320x; 19.4320x over previous
import functools

import jax
import jax.numpy as jnp
from jax import lax
from jax.experimental import pallas as pl
from jax.experimental.pallas import tpu as pltpu
from jax.experimental.pallas import tpu_sc as plsc

BB = 32
GG = 10000
SS = 16
LL = 4
SL = SS * LL
GAMMA = 0.001
INV_GAMMA = 1.0 / GAMMA

NC, NS = 2, 16
NW = NC * NS
GPW = 320
GPAD = NW * GPW
NGB = GPW // 16

_LN2 = 0.6931471805599453


def _vlog(t):
    bits = lax.bitcast_convert_type(t, jnp.int32)
    e = ((bits >> 23) - 127).astype(jnp.float32)
    m = lax.bitcast_convert_type(
        (bits & jnp.int32(0x007FFFFF)) | jnp.int32(0x3F800000), jnp.float32)
    z = (m - 1.0) / (m + 1.0)
    z2 = z * z
    logm = 2.0 * z * (1.0 + z2 * (jnp.float32(1.0 / 3.0) + z2 * jnp.float32(0.2)))
    return e * jnp.float32(_LN2) + logm


def _make_sc_call(interpret=False):
    mesh = plsc.VectorSubcoreMesh(
        core_axis_name="c", subcore_axis_name="s",
        num_cores=NC, num_subcores=NS)

    @functools.partial(
        pl.kernel,
        interpret=interpret,
        out_type=jax.ShapeDtypeStruct((BB * GPAD,), jnp.float32),
        mesh=mesh,
        compiler_params=pltpu.CompilerParams(needs_layout_passes=False),
        scratch_types=[
            pltpu.VMEM((SL * GPW,), jnp.int32),
            pltpu.VMEM((GG,), jnp.float32),
            pltpu.VMEM((GPW,), jnp.float32),
        ],
    )
    def sc_clause(x_hbm, idx_hbm, out_hbm, idx_v, xrow_v, orow_v):
        wid = lax.axis_index("s") * NC + lax.axis_index("c")
        pltpu.sync_copy(idx_hbm.at[pl.ds(wid * (SL * GPW), SL * GPW)], idx_v)

        def b_body(b, carry):
            pltpu.sync_copy(x_hbm.at[pl.ds(b * GG, GG)], xrow_v)

            def gb_body(gb, inner):
                col = gb * 16
                a_list = []
                q_list = []
                for s in range(SS):
                    vs = []
                    for l in range(LL):
                        iv = idx_v[pl.ds((s * LL + l) * GPW + col, 16)]
                        vs.append(plsc.load_gather(xrow_v, [iv]))
                    a = jnp.minimum(jnp.minimum(vs[0], vs[1]),
                                    jnp.minimum(vs[2], vs[3]))
                    q = (jnp.exp((a - vs[0]) * INV_GAMMA)
                         + jnp.exp((a - vs[1]) * INV_GAMMA)
                         + jnp.exp((a - vs[2]) * INV_GAMMA)
                         + jnp.exp((a - vs[3]) * INV_GAMMA))
                    a_list.append(a)
                    q_list.append(q)
                m01 = [jnp.maximum(a_list[2 * i], a_list[2 * i + 1])
                       for i in range(8)]
                m2 = [jnp.maximum(m01[2 * i], m01[2 * i + 1]) for i in range(4)]
                m3 = [jnp.maximum(m2[0], m2[1]), jnp.maximum(m2[2], m2[3])]
                big_m = jnp.maximum(m3[0], m3[1])
                t = jnp.exp((a_list[0] - big_m) * INV_GAMMA) / q_list[0]
                for s in range(1, SS):
                    t = t + jnp.exp((a_list[s] - big_m) * INV_GAMMA) / q_list[s]
                c = big_m + GAMMA * _vlog(t)
                orow_v[pl.ds(col, 16)] = c
                return inner

            lax.fori_loop(0, NGB, gb_body, 0)
            pltpu.sync_copy(
                orow_v, out_hbm.at[pl.ds(b * GPAD + wid * GPW, GPW)])
            return carry

        lax.fori_loop(0, BB, b_body, 0)

    return sc_clause


_SC_CALL_CACHE = []


def kernel(x, I_i):
    if not _SC_CALL_CACHE:
        _SC_CALL_CACHE.append(_make_sc_call())
    sc_clause = _SC_CALL_CACHE[0]
    idx = I_i.reshape(GG, SL).astype(jnp.int32)
    idx = jnp.pad(idx, ((0, GPAD - GG), (0, 0)))
    idx = idx.reshape(NW, GPW, SL).transpose(0, 2, 1).reshape(-1)
    out = sc_clause(x.reshape(-1), idx)
    return out.reshape(BB, GPAD)[:, :GG]
